# baseline (speedup 1.0000x reference)
import functools

import jax
import jax.numpy as jnp
from jax import lax
from jax.experimental import pallas as pl
from jax.experimental.pallas import tpu as pltpu
from jax.experimental.pallas import tpu_sc as plsc

N = 10000
E = 320000
D = 128
NC = 2
NS = 16
K = 32
CHUNKS = 320
PH = 80
CHF = 480
CHS = 160
EP = NS * (CHF + CHS) * K
RPS = 640
NR = NS * RPS
RB = 1024
GRID = NR // RB

_mesh = plsc.VectorSubcoreMesh(core_axis_name="core", subcore_axis_name="subcore")


def _sc_degree(dst4, onesD, zerosD):
    @functools.partial(
        pl.kernel,
        out_type=jax.ShapeDtypeStruct((NC, NR, D), jnp.float32),
        mesh=_mesh,
        scratch_types=[
            pltpu.VMEM((CHUNKS, K), jnp.int32),
            pltpu.VMEM((K, D), jnp.float32),
            pltpu.VMEM_SHARED((NR, D), jnp.float32),
            pltpu.SemaphoreType.DMA,
        ],
    )
    def deg_kernel(dst_hbm, ones_hbm, zeros_hbm, out_hbm, didx, ones_v, acc,
                   ssem):
        c = lax.axis_index("core")
        s = lax.axis_index("subcore")
        pltpu.sync_copy(zeros_hbm, acc.at[pl.ds(s * RPS, RPS)])
        pltpu.sync_copy(ones_hbm, ones_v)
        pltpu.sync_copy(dst_hbm.at[c, s], didx)
        plsc.subcore_barrier()

        @pl.loop(0, CHUNKS)
        def _(j):
            pltpu.async_copy(ones_v, acc.at[didx.at[j]], ssem, add=True)

        @pl.loop(0, CHUNKS)
        def _(j):
            pltpu.make_async_copy(ones_v, acc.at[didx.at[j]], ssem).wait()

        plsc.subcore_barrier()
        pltpu.sync_copy(acc.at[pl.ds(s * RPS, RPS)],
                        out_hbm.at[c, pl.ds(s * RPS, RPS)])

    return deg_kernel(dst4, onesD, zerosD)


def _sc_segsum(g, srcA, dstA, srcB, dstB, zerosD):
    nb = 5

    @functools.partial(
        pl.kernel,
        out_type=jax.ShapeDtypeStruct((NC, NR, D), jnp.float32),
        mesh=_mesh,
        scratch_types=[
            pltpu.VMEM((2, PH, K), jnp.int32),
            pltpu.VMEM((nb, K, D), jnp.float32),
            pltpu.VMEM_SHARED((NR, D), jnp.float32),
            pltpu.SemaphoreType.DMA((nb,)),
            pltpu.SemaphoreType.DMA((nb,)),
        ],
    )
    def seg_kernel(g_hbm, srcA_hbm, dstA_hbm, srcB_hbm, dstB_hbm, zeros_hbm,
                   out_hbm, eidx, rows, acc, gsem, ssem):
        c = lax.axis_index("core")
        s = lax.axis_index("subcore")
        sidx, didx = eidx.at[0], eidx.at[1]
        pltpu.sync_copy(zeros_hbm, acc.at[pl.ds(s * RPS, RPS)])
        plsc.subcore_barrier()

        def run(src_hbm, dst_hbm, nch):
            for p in range(nch // PH):
                pltpu.sync_copy(src_hbm.at[s, pl.ds(p * PH, PH)], sidx)
                pltpu.sync_copy(dst_hbm.at[s, pl.ds(p * PH, PH)], didx)
                for b in (3, 4):
                    pltpu.sync_copy(zeros_hbm.at[pl.ds(0, K)], rows.at[b])
                    pltpu.async_copy(rows.at[b], acc.at[didx.at[0]],
                                     ssem.at[b], add=True)
                for b in range(3):
                    pltpu.async_copy(g_hbm.at[sidx.at[b]], rows.at[b],
                                     gsem.at[b])

                @pl.loop(0, PH, step=nb)
                def _(j0):
                    for b in range(nb):
                        j = j0 + b
                        bp = (b + 3) % nb
                        pltpu.make_async_copy(g_hbm.at[sidx.at[j]],
                                              rows.at[b], gsem.at[b]).wait()
                        jn = lax.rem(j + 3, PH)
                        pltpu.make_async_copy(rows.at[bp],
                                              acc.at[didx.at[j]],
                                              ssem.at[bp]).wait()
                        pltpu.async_copy(g_hbm.at[sidx.at[jn]], rows.at[bp],
                                         gsem.at[bp])
                        pltpu.async_copy(rows.at[b], acc.at[didx.at[j]],
                                         ssem.at[b], add=True)

                for b in (3, 4):
                    pltpu.make_async_copy(rows.at[b], acc.at[didx.at[0]],
                                          ssem.at[b]).wait()
                for b in range(3):
                    pltpu.make_async_copy(g_hbm.at[sidx.at[b]], rows.at[b],
                                          gsem.at[b]).wait()

        @pl.when(c == 0)
        def _():
            run(srcA_hbm, dstA_hbm, CHF)

        @pl.when(c == 1)
        def _():
            run(srcB_hbm, dstB_hbm, CHS)

        plsc.subcore_barrier()
        pltpu.sync_copy(acc.at[pl.ds(s * RPS, RPS)],
                        out_hbm.at[c, pl.ds(s * RPS, RPS)])

    return seg_kernel(g, srcA, dstA, srcB, dstB, zerosD)


def _first_body(x_ref, w_ref, deg_ref, g_ref, dinv_ref):
    d16 = deg_ref[0, :, 0:1] + deg_ref[1, :, 0:1]
    dinv = 1.0 / jnp.sqrt(d16 + 1.0)
    dinv_ref[...] = jnp.broadcast_to(dinv, (RB, D))
    g_ref[...] = jnp.dot(dinv * x_ref[...], w_ref[...],
                         preferred_element_type=jnp.float32)


def _tc_first(xp, W1, degp):
    return pl.pallas_call(
        _first_body,
        grid=(GRID,),
        in_specs=[
            pl.BlockSpec((RB, D), lambda i: (i, 0)),
            pl.BlockSpec((D, D), lambda i: (0, 0)),
            pl.BlockSpec((NC, RB, D), lambda i: (0, i, 0)),
        ],
        out_specs=[
            pl.BlockSpec((RB, D), lambda i: (i, 0)),
            pl.BlockSpec((RB, D), lambda i: (i, 0)),
        ],
        out_shape=[
            jax.ShapeDtypeStruct((NR, D), jnp.float32),
            jax.ShapeDtypeStruct((NR, D), jnp.float32),
        ],
    )(xp, W1, degp)


def _mid_body(p_ref, g_ref, dinv_ref, b_ref, w_ref, o_ref):
    dinv = dinv_ref[...]
    s = p_ref[0] + p_ref[1] + g_ref[...]
    a = jnp.maximum(dinv * s + b_ref[...], 0.0)
    o_ref[...] = jnp.dot(dinv * a, w_ref[...],
                         preferred_element_type=jnp.float32)


def _tc_mid(p, g, dinvb, b, Wn):
    return pl.pallas_call(
        _mid_body,
        grid=(GRID,),
        in_specs=[
            pl.BlockSpec((NC, RB, D), lambda i: (0, i, 0)),
            pl.BlockSpec((RB, D), lambda i: (i, 0)),
            pl.BlockSpec((RB, D), lambda i: (i, 0)),
            pl.BlockSpec((1, D), lambda i: (0, 0)),
            pl.BlockSpec((D, D), lambda i: (0, 0)),
        ],
        out_specs=pl.BlockSpec((RB, D), lambda i: (i, 0)),
        out_shape=jax.ShapeDtypeStruct((NR, D), jnp.float32),
    )(p, g, dinvb, b, Wn)


def _last_body(p_ref, g_ref, dinv_ref, b_ref, o_ref):
    s = p_ref[0] + p_ref[1] + g_ref[...]
    o_ref[...] = dinv_ref[...] * s + b_ref[...]


def _tc_last(p, g, dinvb, b):
    return pl.pallas_call(
        _last_body,
        grid=(GRID,),
        in_specs=[
            pl.BlockSpec((NC, RB, D), lambda i: (0, i, 0)),
            pl.BlockSpec((RB, D), lambda i: (i, 0)),
            pl.BlockSpec((RB, D), lambda i: (i, 0)),
            pl.BlockSpec((1, D), lambda i: (0, 0)),
        ],
        out_specs=pl.BlockSpec((RB, D), lambda i: (i, 0)),
        out_shape=jax.ShapeDtypeStruct((NR, D), jnp.float32),
    )(p, g, dinvb, b)


def kernel(x, edge_index, W1, b1, W2, b2, W3, b3):
    src = edge_index[0].astype(jnp.int32)
    dst = edge_index[1].astype(jnp.int32)
    pad = EP - E
    pad_dst = (N + jnp.arange(pad, dtype=jnp.int32) % (NR - N))
    src_flat = jnp.concatenate([src, jnp.zeros((pad,), jnp.int32)])
    dst_flat = jnp.concatenate([dst, pad_dst])
    src4 = src_flat.reshape(NC, NS, CHUNKS, K)
    dst4 = dst_flat.reshape(NC, NS, CHUNKS, K)
    cut = NS * CHF * K
    srcA = src_flat[:cut].reshape(NS, CHF, K)
    dstA = dst_flat[:cut].reshape(NS, CHF, K)
    srcB = src_flat[cut:].reshape(NS, CHS, K)
    dstB = dst_flat[cut:].reshape(NS, CHS, K)
    xp = jnp.zeros((NR, D), jnp.float32).at[:N].set(x)
    onesD = jnp.ones((K, D), jnp.float32)
    zerosD = jnp.zeros((RPS, D), jnp.float32)
    b1r, b2r, b3r = (b.reshape(1, D) for b in (b1, b2, b3))

    degp = _sc_degree(dst4, onesD, zerosD)
    g1, dinvb = _tc_first(xp, W1, degp)
    p1 = _sc_segsum(g1, srcA, dstA, srcB, dstB, zerosD)
    g2 = _tc_mid(p1, g1, dinvb, b1r, W2)
    p2 = _sc_segsum(g2, srcA, dstA, srcB, dstB, zerosD)
    g3 = _tc_mid(p2, g2, dinvb, b2r, W3)
    p3 = _sc_segsum(g3, srcA, dstA, srcB, dstB, zerosD)
    out = _tc_last(p3, g3, dinvb, b3r)
    return out[:N]


# speedup vs baseline: 1.0641x; 1.0641x over previous
import functools

import jax
import jax.numpy as jnp
from jax import lax
from jax.experimental import pallas as pl
from jax.experimental.pallas import tpu as pltpu
from jax.experimental.pallas import tpu_sc as plsc

N = 10000
E = 320000
D = 128
NC = 2
NS = 16
K = 64
CHUNKS = 160
PH = 80
CHF = 240
CHS = 80
EP = NS * (CHF + CHS) * K
RPS = 640
NR = NS * RPS
RB = 1024
GRID = NR // RB

_mesh = plsc.VectorSubcoreMesh(core_axis_name="core", subcore_axis_name="subcore")


def _sc_degree(dst4, onesD, zerosD):
    @functools.partial(
        pl.kernel,
        out_type=jax.ShapeDtypeStruct((NC, NR, D), jnp.float32),
        mesh=_mesh,
        scratch_types=[
            pltpu.VMEM((CHUNKS, K), jnp.int32),
            pltpu.VMEM((K, D), jnp.float32),
            pltpu.VMEM_SHARED((NR, D), jnp.float32),
            pltpu.SemaphoreType.DMA,
        ],
    )
    def deg_kernel(dst_hbm, ones_hbm, zeros_hbm, out_hbm, didx, ones_v, acc,
                   ssem):
        c = lax.axis_index("core")
        s = lax.axis_index("subcore")
        pltpu.sync_copy(zeros_hbm, acc.at[pl.ds(s * RPS, RPS)])
        pltpu.sync_copy(ones_hbm, ones_v)
        pltpu.sync_copy(dst_hbm.at[c, s], didx)
        plsc.subcore_barrier()

        @pl.loop(0, CHUNKS)
        def _(j):
            pltpu.async_copy(ones_v, acc.at[didx.at[j]], ssem, add=True)

        @pl.loop(0, CHUNKS)
        def _(j):
            pltpu.make_async_copy(ones_v, acc.at[didx.at[j]], ssem).wait()

        plsc.subcore_barrier()
        pltpu.sync_copy(acc.at[pl.ds(s * RPS, RPS)],
                        out_hbm.at[c, pl.ds(s * RPS, RPS)])

    return deg_kernel(dst4, onesD, zerosD)


def _sc_segsum(g, srcA, dstA, srcB, dstB, zerosD):
    nb = 2

    @functools.partial(
        pl.kernel,
        out_type=jax.ShapeDtypeStruct((NC, NR, D), jnp.float32),
        mesh=_mesh,
        scratch_types=[
            pltpu.VMEM((2, PH, K), jnp.int32),
            pltpu.VMEM((nb, K, D), jnp.float32),
            pltpu.VMEM_SHARED((NR, D), jnp.float32),
            pltpu.SemaphoreType.DMA((nb,)),
            pltpu.SemaphoreType.DMA((nb,)),
        ],
    )
    def seg_kernel(g_hbm, srcA_hbm, dstA_hbm, srcB_hbm, dstB_hbm, zeros_hbm,
                   out_hbm, eidx, rows, acc, gsem, ssem):
        c = lax.axis_index("core")
        s = lax.axis_index("subcore")
        sidx, didx = eidx.at[0], eidx.at[1]
        pltpu.sync_copy(zeros_hbm, acc.at[pl.ds(s * RPS, RPS)])
        plsc.subcore_barrier()

        def run(src_hbm, dst_hbm, nch):
            for p in range(nch // PH):
                pltpu.sync_copy(src_hbm.at[s, pl.ds(p * PH, PH)], sidx)
                pltpu.sync_copy(dst_hbm.at[s, pl.ds(p * PH, PH)], didx)
                pltpu.sync_copy(zeros_hbm.at[pl.ds(0, K)], rows.at[1])
                pltpu.async_copy(rows.at[1], acc.at[didx.at[0]], ssem.at[1],
                                 add=True)
                pltpu.async_copy(g_hbm.at[sidx.at[0]], rows.at[0],
                                 gsem.at[0])

                @pl.loop(0, PH, step=nb)
                def _(j0):
                    for b in range(nb):
                        j = j0 + b
                        bn = 1 - b
                        pltpu.make_async_copy(g_hbm.at[sidx.at[j]],
                                              rows.at[b], gsem.at[b]).wait()
                        jn = lax.rem(j + 1, PH)
                        pltpu.make_async_copy(rows.at[bn],
                                              acc.at[didx.at[j]],
                                              ssem.at[bn]).wait()
                        pltpu.async_copy(g_hbm.at[sidx.at[jn]], rows.at[bn],
                                         gsem.at[bn])
                        pltpu.async_copy(rows.at[b], acc.at[didx.at[j]],
                                         ssem.at[b], add=True)

                pltpu.make_async_copy(rows.at[1], acc.at[didx.at[0]],
                                      ssem.at[1]).wait()
                pltpu.make_async_copy(g_hbm.at[sidx.at[0]], rows.at[0],
                                      gsem.at[0]).wait()

        @pl.when(c == 0)
        def _():
            run(srcA_hbm, dstA_hbm, CHF)

        @pl.when(c == 1)
        def _():
            run(srcB_hbm, dstB_hbm, CHS)

        plsc.subcore_barrier()
        pltpu.sync_copy(acc.at[pl.ds(s * RPS, RPS)],
                        out_hbm.at[c, pl.ds(s * RPS, RPS)])

    return seg_kernel(g, srcA, dstA, srcB, dstB, zerosD)


def _first_body(x_ref, w_ref, deg_ref, g_ref, dinv_ref):
    d16 = deg_ref[0, :, 0:1] + deg_ref[1, :, 0:1]
    dinv = 1.0 / jnp.sqrt(d16 + 1.0)
    dinv_ref[...] = jnp.broadcast_to(dinv, (RB, D))
    g_ref[...] = jnp.dot(dinv * x_ref[...], w_ref[...],
                         preferred_element_type=jnp.float32)


def _tc_first(xp, W1, degp):
    return pl.pallas_call(
        _first_body,
        grid=(GRID,),
        in_specs=[
            pl.BlockSpec((RB, D), lambda i: (i, 0)),
            pl.BlockSpec((D, D), lambda i: (0, 0)),
            pl.BlockSpec((NC, RB, D), lambda i: (0, i, 0)),
        ],
        out_specs=[
            pl.BlockSpec((RB, D), lambda i: (i, 0)),
            pl.BlockSpec((RB, D), lambda i: (i, 0)),
        ],
        out_shape=[
            jax.ShapeDtypeStruct((NR, D), jnp.float32),
            jax.ShapeDtypeStruct((NR, D), jnp.float32),
        ],
    )(xp, W1, degp)


def _mid_body(p_ref, g_ref, dinv_ref, b_ref, w_ref, o_ref):
    dinv = dinv_ref[...]
    s = p_ref[0] + p_ref[1] + g_ref[...]
    a = jnp.maximum(dinv * s + b_ref[...], 0.0)
    o_ref[...] = jnp.dot(dinv * a, w_ref[...],
                         preferred_element_type=jnp.float32)


def _tc_mid(p, g, dinvb, b, Wn):
    return pl.pallas_call(
        _mid_body,
        grid=(GRID,),
        in_specs=[
            pl.BlockSpec((NC, RB, D), lambda i: (0, i, 0)),
            pl.BlockSpec((RB, D), lambda i: (i, 0)),
            pl.BlockSpec((RB, D), lambda i: (i, 0)),
            pl.BlockSpec((1, D), lambda i: (0, 0)),
            pl.BlockSpec((D, D), lambda i: (0, 0)),
        ],
        out_specs=pl.BlockSpec((RB, D), lambda i: (i, 0)),
        out_shape=jax.ShapeDtypeStruct((NR, D), jnp.float32),
    )(p, g, dinvb, b, Wn)


def _last_body(p_ref, g_ref, dinv_ref, b_ref, o_ref):
    s = p_ref[0] + p_ref[1] + g_ref[...]
    o_ref[...] = dinv_ref[...] * s + b_ref[...]


def _tc_last(p, g, dinvb, b):
    return pl.pallas_call(
        _last_body,
        grid=(GRID,),
        in_specs=[
            pl.BlockSpec((NC, RB, D), lambda i: (0, i, 0)),
            pl.BlockSpec((RB, D), lambda i: (i, 0)),
            pl.BlockSpec((RB, D), lambda i: (i, 0)),
            pl.BlockSpec((1, D), lambda i: (0, 0)),
        ],
        out_specs=pl.BlockSpec((RB, D), lambda i: (i, 0)),
        out_shape=jax.ShapeDtypeStruct((NR, D), jnp.float32),
    )(p, g, dinvb, b)


def kernel(x, edge_index, W1, b1, W2, b2, W3, b3):
    src = edge_index[0].astype(jnp.int32)
    dst = edge_index[1].astype(jnp.int32)
    pad = EP - E
    pad_dst = (N + jnp.arange(pad, dtype=jnp.int32) % (NR - N))
    src_flat = jnp.concatenate([src, jnp.zeros((pad,), jnp.int32)])
    dst_flat = jnp.concatenate([dst, pad_dst])
    src4 = src_flat.reshape(NC, NS, CHUNKS, K)
    dst4 = dst_flat.reshape(NC, NS, CHUNKS, K)
    cut = NS * CHF * K
    srcA = src_flat[:cut].reshape(NS, CHF, K)
    dstA = dst_flat[:cut].reshape(NS, CHF, K)
    srcB = src_flat[cut:].reshape(NS, CHS, K)
    dstB = dst_flat[cut:].reshape(NS, CHS, K)
    xp = jnp.zeros((NR, D), jnp.float32).at[:N].set(x)
    onesD = jnp.ones((K, D), jnp.float32)
    zerosD = jnp.zeros((RPS, D), jnp.float32)
    b1r, b2r, b3r = (b.reshape(1, D) for b in (b1, b2, b3))

    degp = _sc_degree(dst4, onesD, zerosD)
    g1, dinvb = _tc_first(xp, W1, degp)
    p1 = _sc_segsum(g1, srcA, dstA, srcB, dstB, zerosD)
    g2 = _tc_mid(p1, g1, dinvb, b1r, W2)
    p2 = _sc_segsum(g2, srcA, dstA, srcB, dstB, zerosD)
    g3 = _tc_mid(p2, g2, dinvb, b2r, W3)
    p3 = _sc_segsum(g3, srcA, dstA, srcB, dstB, zerosD)
    out = _tc_last(p3, g3, dinvb, b3r)
    return out[:N]


# speedup vs baseline: 1.0961x; 1.0301x over previous
import functools

import jax
import jax.numpy as jnp
from jax import lax
from jax.experimental import pallas as pl
from jax.experimental.pallas import tpu as pltpu
from jax.experimental.pallas import tpu_sc as plsc

N = 10000
E = 320000
D = 128
NC = 2
NS = 16
K = 64
CHUNKS = 160
PH = 64
CHF = 256
CHS = 64
EP = NS * (CHF + CHS) * K
RPS = 640
NR = NS * RPS
RB = 1024
GRID = NR // RB

_mesh = plsc.VectorSubcoreMesh(core_axis_name="core", subcore_axis_name="subcore")


def _sc_degree(dst4, onesD, zerosD):
    @functools.partial(
        pl.kernel,
        out_type=jax.ShapeDtypeStruct((NC, NR, D), jnp.float32),
        mesh=_mesh,
        scratch_types=[
            pltpu.VMEM((CHUNKS, K), jnp.int32),
            pltpu.VMEM((K, D), jnp.float32),
            pltpu.VMEM_SHARED((NR, D), jnp.float32),
            pltpu.SemaphoreType.DMA,
        ],
    )
    def deg_kernel(dst_hbm, ones_hbm, zeros_hbm, out_hbm, didx, ones_v, acc,
                   ssem):
        c = lax.axis_index("core")
        s = lax.axis_index("subcore")
        pltpu.sync_copy(zeros_hbm, acc.at[pl.ds(s * RPS, RPS)])
        pltpu.sync_copy(ones_hbm, ones_v)
        pltpu.sync_copy(dst_hbm.at[c, s], didx)
        plsc.subcore_barrier()

        @pl.loop(0, CHUNKS)
        def _(j):
            pltpu.async_copy(ones_v, acc.at[didx.at[j]], ssem, add=True)

        @pl.loop(0, CHUNKS)
        def _(j):
            pltpu.make_async_copy(ones_v, acc.at[didx.at[j]], ssem).wait()

        plsc.subcore_barrier()
        pltpu.sync_copy(acc.at[pl.ds(s * RPS, RPS)],
                        out_hbm.at[c, pl.ds(s * RPS, RPS)])

    return deg_kernel(dst4, onesD, zerosD)


def _sc_segsum(g, srcA, dstA, srcB, dstB, zerosD):
    nb = 2

    @functools.partial(
        pl.kernel,
        out_type=jax.ShapeDtypeStruct((NC, NR, D), jnp.float32),
        mesh=_mesh,
        scratch_types=[
            pltpu.VMEM((2, PH, K), jnp.int32),
            pltpu.VMEM((nb, K, D), jnp.float32),
            pltpu.VMEM_SHARED((NR, D), jnp.float32),
            pltpu.SemaphoreType.DMA((nb,)),
            pltpu.SemaphoreType.DMA((nb,)),
        ],
    )
    def seg_kernel(g_hbm, srcA_hbm, dstA_hbm, srcB_hbm, dstB_hbm, zeros_hbm,
                   out_hbm, eidx, rows, acc, gsem, ssem):
        c = lax.axis_index("core")
        s = lax.axis_index("subcore")
        sidx, didx = eidx.at[0], eidx.at[1]
        pltpu.sync_copy(zeros_hbm, acc.at[pl.ds(s * RPS, RPS)])
        plsc.subcore_barrier()

        def run(src_hbm, dst_hbm, nch):
            for p in range(nch // PH):
                pltpu.sync_copy(src_hbm.at[s, pl.ds(p * PH, PH)], sidx)
                pltpu.sync_copy(dst_hbm.at[s, pl.ds(p * PH, PH)], didx)
                pltpu.sync_copy(zeros_hbm.at[pl.ds(0, K)], rows.at[1])
                pltpu.async_copy(rows.at[1], acc.at[didx.at[0]], ssem.at[1],
                                 add=True)
                pltpu.async_copy(g_hbm.at[sidx.at[0]], rows.at[0],
                                 gsem.at[0])

                @pl.loop(0, PH, step=nb)
                def _(j0):
                    for b in range(nb):
                        j = j0 + b
                        bn = 1 - b
                        pltpu.make_async_copy(g_hbm.at[sidx.at[j]],
                                              rows.at[b], gsem.at[b]).wait()
                        jn = lax.rem(j + 1, PH)
                        pltpu.make_async_copy(rows.at[bn],
                                              acc.at[didx.at[j]],
                                              ssem.at[bn]).wait()
                        pltpu.async_copy(g_hbm.at[sidx.at[jn]], rows.at[bn],
                                         gsem.at[bn])
                        pltpu.async_copy(rows.at[b], acc.at[didx.at[j]],
                                         ssem.at[b], add=True)

                pltpu.make_async_copy(rows.at[1], acc.at[didx.at[0]],
                                      ssem.at[1]).wait()
                pltpu.make_async_copy(g_hbm.at[sidx.at[0]], rows.at[0],
                                      gsem.at[0]).wait()

        @pl.when(c == 0)
        def _():
            run(srcA_hbm, dstA_hbm, CHF)

        @pl.when(c == 1)
        def _():
            run(srcB_hbm, dstB_hbm, CHS)

        plsc.subcore_barrier()
        pltpu.sync_copy(acc.at[pl.ds(s * RPS, RPS)],
                        out_hbm.at[c, pl.ds(s * RPS, RPS)])

    return seg_kernel(g, srcA, dstA, srcB, dstB, zerosD)


def _first_body(x_ref, w_ref, deg_ref, g_ref, dinv_ref):
    d16 = deg_ref[0, :, 0:1] + deg_ref[1, :, 0:1]
    dinv = 1.0 / jnp.sqrt(d16 + 1.0)
    dinv_ref[...] = jnp.broadcast_to(dinv, (RB, D))
    g_ref[...] = jnp.dot(dinv * x_ref[...], w_ref[...],
                         preferred_element_type=jnp.float32)


def _tc_first(xp, W1, degp):
    return pl.pallas_call(
        _first_body,
        grid=(GRID,),
        in_specs=[
            pl.BlockSpec((RB, D), lambda i: (i, 0)),
            pl.BlockSpec((D, D), lambda i: (0, 0)),
            pl.BlockSpec((NC, RB, D), lambda i: (0, i, 0)),
        ],
        out_specs=[
            pl.BlockSpec((RB, D), lambda i: (i, 0)),
            pl.BlockSpec((RB, D), lambda i: (i, 0)),
        ],
        out_shape=[
            jax.ShapeDtypeStruct((NR, D), jnp.float32),
            jax.ShapeDtypeStruct((NR, D), jnp.float32),
        ],
    )(xp, W1, degp)


def _mid_body(p_ref, g_ref, dinv_ref, b_ref, w_ref, o_ref):
    dinv = dinv_ref[...]
    s = p_ref[0] + p_ref[1] + g_ref[...]
    a = jnp.maximum(dinv * s + b_ref[...], 0.0)
    o_ref[...] = jnp.dot(dinv * a, w_ref[...],
                         preferred_element_type=jnp.float32)


def _tc_mid(p, g, dinvb, b, Wn):
    return pl.pallas_call(
        _mid_body,
        grid=(GRID,),
        in_specs=[
            pl.BlockSpec((NC, RB, D), lambda i: (0, i, 0)),
            pl.BlockSpec((RB, D), lambda i: (i, 0)),
            pl.BlockSpec((RB, D), lambda i: (i, 0)),
            pl.BlockSpec((1, D), lambda i: (0, 0)),
            pl.BlockSpec((D, D), lambda i: (0, 0)),
        ],
        out_specs=pl.BlockSpec((RB, D), lambda i: (i, 0)),
        out_shape=jax.ShapeDtypeStruct((NR, D), jnp.float32),
    )(p, g, dinvb, b, Wn)


def _last_body(p_ref, g_ref, dinv_ref, b_ref, o_ref):
    s = p_ref[0] + p_ref[1] + g_ref[...]
    o_ref[...] = dinv_ref[...] * s + b_ref[...]


def _tc_last(p, g, dinvb, b):
    return pl.pallas_call(
        _last_body,
        grid=(GRID,),
        in_specs=[
            pl.BlockSpec((NC, RB, D), lambda i: (0, i, 0)),
            pl.BlockSpec((RB, D), lambda i: (i, 0)),
            pl.BlockSpec((RB, D), lambda i: (i, 0)),
            pl.BlockSpec((1, D), lambda i: (0, 0)),
        ],
        out_specs=pl.BlockSpec((RB, D), lambda i: (i, 0)),
        out_shape=jax.ShapeDtypeStruct((NR, D), jnp.float32),
    )(p, g, dinvb, b)


def kernel(x, edge_index, W1, b1, W2, b2, W3, b3):
    src = edge_index[0].astype(jnp.int32)
    dst = edge_index[1].astype(jnp.int32)
    pad = EP - E
    pad_dst = (N + jnp.arange(pad, dtype=jnp.int32) % (NR - N))
    src_flat = jnp.concatenate([src, jnp.zeros((pad,), jnp.int32)])
    dst_flat = jnp.concatenate([dst, pad_dst])
    src4 = src_flat.reshape(NC, NS, CHUNKS, K)
    dst4 = dst_flat.reshape(NC, NS, CHUNKS, K)
    cut = NS * CHF * K
    srcA = src_flat[:cut].reshape(NS, CHF, K)
    dstA = dst_flat[:cut].reshape(NS, CHF, K)
    srcB = src_flat[cut:].reshape(NS, CHS, K)
    dstB = dst_flat[cut:].reshape(NS, CHS, K)
    xp = jnp.zeros((NR, D), jnp.float32).at[:N].set(x)
    onesD = jnp.ones((K, D), jnp.float32)
    zerosD = jnp.zeros((RPS, D), jnp.float32)
    b1r, b2r, b3r = (b.reshape(1, D) for b in (b1, b2, b3))

    degp = _sc_degree(dst4, onesD, zerosD)
    g1, dinvb = _tc_first(xp, W1, degp)
    p1 = _sc_segsum(g1, srcA, dstA, srcB, dstB, zerosD)
    g2 = _tc_mid(p1, g1, dinvb, b1r, W2)
    p2 = _sc_segsum(g2, srcA, dstA, srcB, dstB, zerosD)
    g3 = _tc_mid(p2, g2, dinvb, b2r, W3)
    p3 = _sc_segsum(g3, srcA, dstA, srcB, dstB, zerosD)
    out = _tc_last(p3, g3, dinvb, b3r)
    return out[:N]
